# baseline (speedup 1.0000x reference)
import jax
import jax.numpy as jnp
from jax.experimental import pallas as pl
from jax.experimental.pallas import tpu as pltpu

N = 4096
D = 128
H = 64
G = 128
C = 10

BM = 512
M = N // BM


def _hamilton(kernel):
    r, i, j, k = jnp.split(kernel, 4, axis=1)
    r2 = jnp.concatenate([r, -i, -j, -k], axis=0)
    i2 = jnp.concatenate([i, r, -k, j], axis=0)
    j2 = jnp.concatenate([j, k, r, -i], axis=0)
    k2 = jnp.concatenate([k, -j, i, r], axis=0)
    return jnp.concatenate([r2, i2, j2, k2], axis=1)


def _qgnn_kernel(adj_ref, x0_ref, gp_ref, h0_ref, h1_ref, pw0_ref, pw1_ref,
                 pb_ref, scores_ref, adj_sc, sup_sc, x_sc):
    l = pl.program_id(0)
    m = pl.program_id(1)

    @pl.when((l == 0) & (m == 0))
    def _():
        scores_ref[...] = jnp.broadcast_to(pb_ref[...], scores_ref.shape)
        sup_sc[...] = jnp.dot(
            x0_ref[...], h0_ref[...],
            preferred_element_type=jnp.float32).astype(jnp.bfloat16)

    @pl.when((l == 1) & (m == 0))
    def _():
        sup_sc[...] = jnp.dot(
            x_sc[...], h1_ref[...],
            preferred_element_type=jnp.float32).astype(jnp.bfloat16)

    @pl.when(l == 0)
    def _():
        adj_sc[m] = adj_ref[...].astype(jnp.bfloat16)

    z = jnp.dot(adj_sc[m], sup_sc[...], preferred_element_type=jnp.float32)
    xt = jnp.tanh(z).astype(jnp.bfloat16)

    @pl.when(l == 0)
    def _():
        x_sc[pl.ds(m * BM, BM), :] = xt

    ge = jnp.dot(gp_ref[...], xt, preferred_element_type=jnp.float32)
    pw = jnp.where(l == 0, pw0_ref[...], pw1_ref[...])
    scores_ref[...] += jnp.dot(ge.astype(jnp.bfloat16), pw,
                               preferred_element_type=jnp.float32)


def kernel(Adj_block, X_concat, graph_pool, W0, W1, P0_w, P0_b, P1_w, P1_b):
    h0 = _hamilton(W0).astype(jnp.bfloat16)
    h1 = _hamilton(W1).astype(jnp.bfloat16)
    x0 = X_concat.astype(jnp.bfloat16)
    gp = graph_pool.astype(jnp.bfloat16)
    pw0 = P0_w.astype(jnp.bfloat16)
    pw1 = P1_w.astype(jnp.bfloat16)
    pb = (P0_b + P1_b).reshape(1, C)

    scores = pl.pallas_call(
        _qgnn_kernel,
        grid=(2, M),
        in_specs=[
            pl.BlockSpec((BM, N), lambda l, m: (jnp.where(l == 0, m, 0), 0)),
            pl.BlockSpec((N, D), lambda l, m: (0, 0)),
            pl.BlockSpec((G, BM), lambda l, m: (0, m)),
            pl.BlockSpec((D, H), lambda l, m: (0, 0)),
            pl.BlockSpec((H, H), lambda l, m: (0, 0)),
            pl.BlockSpec((H, C), lambda l, m: (0, 0)),
            pl.BlockSpec((H, C), lambda l, m: (0, 0)),
            pl.BlockSpec((1, C), lambda l, m: (0, 0)),
        ],
        out_specs=pl.BlockSpec((G, C), lambda l, m: (0, 0)),
        out_shape=jax.ShapeDtypeStruct((G, C), jnp.float32),
        scratch_shapes=[
            pltpu.VMEM((M, BM, N), jnp.bfloat16),
            pltpu.VMEM((N, H), jnp.bfloat16),
            pltpu.VMEM((N, H), jnp.bfloat16),
        ],
        compiler_params=pltpu.CompilerParams(
            dimension_semantics=("arbitrary", "arbitrary"),
        ),
    )(Adj_block, x0, gp, h0, h1, pw0, pw1, pb)
    return scores


# speedup vs baseline: 1.0691x; 1.0691x over previous
import jax
import jax.numpy as jnp
from jax.experimental import pallas as pl
from jax.experimental.pallas import tpu as pltpu

N = 4096
D = 128
H = 64
G = 128
C = 10

BM = 512
M = N // BM


def _hamilton(kernel):
    r, i, j, k = jnp.split(kernel, 4, axis=1)
    r2 = jnp.concatenate([r, -i, -j, -k], axis=0)
    i2 = jnp.concatenate([i, r, -k, j], axis=0)
    j2 = jnp.concatenate([j, k, r, -i], axis=0)
    k2 = jnp.concatenate([k, -j, i, r], axis=0)
    return jnp.concatenate([r2, i2, j2, k2], axis=1)


def _qgnn_kernel(adj_ref, sup0_ref, gp_ref, h1_ref, pw0_ref, pw1_ref,
                 pb_ref, scores_ref, adj_sc, sup_sc, x_sc, xt_sc):
    l = pl.program_id(0)
    m = pl.program_id(1)

    @pl.when((l == 0) & (m == 0))
    def _():
        scores_ref[...] = jnp.broadcast_to(pb_ref[...], scores_ref.shape)
        sup_sc[...] = sup0_ref[...]

    @pl.when((l == 1) & (m == 0))
    def _():
        sup_sc[...] = jnp.dot(
            x_sc[...], h1_ref[...],
            preferred_element_type=jnp.float32).astype(jnp.bfloat16)

    @pl.when(l == 0)
    def _():
        a_bf = adj_ref[...].astype(jnp.bfloat16)
        adj_sc[m] = a_bf
        xt = jnp.tanh(jnp.dot(a_bf, sup_sc[...],
                              preferred_element_type=jnp.float32))
        xt_bf = xt.astype(jnp.bfloat16)
        x_sc[pl.ds(m * BM, BM), :] = xt_bf
        xt_sc[...] = xt_bf

    @pl.when(l == 1)
    def _():
        xt = jnp.tanh(jnp.dot(adj_sc[m], sup_sc[...],
                              preferred_element_type=jnp.float32))
        xt_sc[...] = xt.astype(jnp.bfloat16)

    xt = xt_sc[...]
    ge = jnp.dot(gp_ref[...], xt, preferred_element_type=jnp.float32)
    pw = jnp.where(l == 0, pw0_ref[...], pw1_ref[...])
    scores_ref[...] += jnp.dot(ge.astype(jnp.bfloat16), pw,
                               preferred_element_type=jnp.float32)


def kernel(Adj_block, X_concat, graph_pool, W0, W1, P0_w, P0_b, P1_w, P1_b):
    h0 = _hamilton(W0)
    h1 = _hamilton(W1).astype(jnp.bfloat16)
    sup0 = jnp.dot(X_concat, h0).astype(jnp.bfloat16)
    gp = graph_pool.astype(jnp.bfloat16)
    pw0 = P0_w.astype(jnp.bfloat16)
    pw1 = P1_w.astype(jnp.bfloat16)
    pb = (P0_b + P1_b).reshape(1, C)

    scores = pl.pallas_call(
        _qgnn_kernel,
        grid=(2, M),
        in_specs=[
            pl.BlockSpec((BM, N), lambda l, m: (jnp.where(l == 0, m, 0), 0)),
            pl.BlockSpec((N, H), lambda l, m: (0, 0)),
            pl.BlockSpec((G, BM), lambda l, m: (0, m)),
            pl.BlockSpec((H, H), lambda l, m: (0, 0)),
            pl.BlockSpec((H, C), lambda l, m: (0, 0)),
            pl.BlockSpec((H, C), lambda l, m: (0, 0)),
            pl.BlockSpec((1, C), lambda l, m: (0, 0)),
        ],
        out_specs=pl.BlockSpec((G, C), lambda l, m: (0, 0)),
        out_shape=jax.ShapeDtypeStruct((G, C), jnp.float32),
        scratch_shapes=[
            pltpu.VMEM((M, BM, N), jnp.bfloat16),
            pltpu.VMEM((N, H), jnp.bfloat16),
            pltpu.VMEM((N, H), jnp.bfloat16),
            pltpu.VMEM((BM, H), jnp.bfloat16),
        ],
        compiler_params=pltpu.CompilerParams(
            dimension_semantics=("arbitrary", "arbitrary"),
        ),
    )(Adj_block, sup0, gp, h1, pw0, pw1, pb)
    return scores


# speedup vs baseline: 1.0780x; 1.0083x over previous
import jax
import jax.numpy as jnp
from jax.experimental import pallas as pl
from jax.experimental.pallas import tpu as pltpu

N = 4096
D = 128
H = 64
G = 128
C = 10

BM = 512
M = N // BM


def _hamilton(kernel):
    r, i, j, k = jnp.split(kernel, 4, axis=1)
    r2 = jnp.concatenate([r, -i, -j, -k], axis=0)
    i2 = jnp.concatenate([i, r, -k, j], axis=0)
    j2 = jnp.concatenate([j, k, r, -i], axis=0)
    k2 = jnp.concatenate([k, -j, i, r], axis=0)
    return jnp.concatenate([r2, i2, j2, k2], axis=1)


def _qgnn_kernel(adj_ref, sup0_ref, gp_ref, h1_ref, pw0_ref, pw1_ref,
                 pb_ref, scores_ref, adj_sc, sup_sc, x_sc):
    l = pl.program_id(0)
    m = pl.program_id(1)

    def pool_head(xt_bf, pw_ref):
        ge = jnp.dot(gp_ref[...], xt_bf, preferred_element_type=jnp.float32)
        scores_ref[...] += jnp.dot(ge.astype(jnp.bfloat16), pw_ref[...],
                                   preferred_element_type=jnp.float32)

    @pl.when((l == 0) & (m == 0))
    def _():
        scores_ref[...] = jnp.broadcast_to(pb_ref[...], scores_ref.shape)
        sup_sc[...] = sup0_ref[...]

    @pl.when((l == 1) & (m == 0))
    def _():
        sup_sc[...] = jnp.dot(
            x_sc[...], h1_ref[...],
            preferred_element_type=jnp.float32).astype(jnp.bfloat16)

    @pl.when(l == 0)
    def _():
        a_bf = adj_ref[...].astype(jnp.bfloat16)
        adj_sc[m] = a_bf
        xt = jnp.tanh(jnp.dot(a_bf, sup_sc[...],
                              preferred_element_type=jnp.float32))
        xt_bf = xt.astype(jnp.bfloat16)
        x_sc[pl.ds(m * BM, BM), :] = xt_bf
        pool_head(xt_bf, pw0_ref)

    @pl.when(l == 1)
    def _():
        xt = jnp.tanh(jnp.dot(adj_sc[m], sup_sc[...],
                              preferred_element_type=jnp.float32))
        pool_head(xt.astype(jnp.bfloat16), pw1_ref)


def kernel(Adj_block, X_concat, graph_pool, W0, W1, P0_w, P0_b, P1_w, P1_b):
    h0 = _hamilton(W0)
    h1 = _hamilton(W1).astype(jnp.bfloat16)
    sup0 = jnp.dot(X_concat, h0).astype(jnp.bfloat16)
    gp = graph_pool.astype(jnp.bfloat16)
    pw0 = P0_w.astype(jnp.bfloat16)
    pw1 = P1_w.astype(jnp.bfloat16)
    pb = (P0_b + P1_b).reshape(1, C)

    scores = pl.pallas_call(
        _qgnn_kernel,
        grid=(2, M),
        in_specs=[
            pl.BlockSpec((BM, N), lambda l, m: (jnp.where(l == 0, m, 0), 0)),
            pl.BlockSpec((N, H), lambda l, m: (0, 0)),
            pl.BlockSpec((G, BM), lambda l, m: (0, m)),
            pl.BlockSpec((H, H), lambda l, m: (0, 0)),
            pl.BlockSpec((H, C), lambda l, m: (0, 0)),
            pl.BlockSpec((H, C), lambda l, m: (0, 0)),
            pl.BlockSpec((1, C), lambda l, m: (0, 0)),
        ],
        out_specs=pl.BlockSpec((G, C), lambda l, m: (0, 0)),
        out_shape=jax.ShapeDtypeStruct((G, C), jnp.float32),
        scratch_shapes=[
            pltpu.VMEM((M, BM, N), jnp.bfloat16),
            pltpu.VMEM((N, H), jnp.bfloat16),
            pltpu.VMEM((N, H), jnp.bfloat16),
        ],
        compiler_params=pltpu.CompilerParams(
            dimension_semantics=("arbitrary", "arbitrary"),
        ),
    )(Adj_block, sup0, gp, h1, pw0, pw1, pb)
    return scores


# speedup vs baseline: 1.1729x; 1.0880x over previous
import jax
import jax.numpy as jnp
from jax.experimental import pallas as pl
from jax.experimental.pallas import tpu as pltpu

N = 4096
D = 128
H = 64
G = 128
C = 10

BM = 512
M = N // BM


def _hamilton(kernel):
    r, i, j, k = jnp.split(kernel, 4, axis=1)
    r2 = jnp.concatenate([r, -i, -j, -k], axis=0)
    i2 = jnp.concatenate([i, r, -k, j], axis=0)
    j2 = jnp.concatenate([j, k, r, -i], axis=0)
    k2 = jnp.concatenate([k, -j, i, r], axis=0)
    return jnp.concatenate([r2, i2, j2, k2], axis=1)


def _qgnn_kernel(adj_ref, sup0_ref, gp_ref, h1_ref, pw0_ref, pw1_ref,
                 pb_ref, scores_ref, adj_sc, sup1_sc, x_sc, s0_sc, s1_sc):
    l = pl.program_id(0)
    m = pl.program_id(1)

    @pl.when((l == 1) & (m == 0))
    def _():
        sup1_sc[...] = jnp.dot(
            x_sc[...], h1_ref[...],
            preferred_element_type=jnp.float32).astype(jnp.bfloat16)

    @pl.when(l == 1)
    def _():
        xt = jnp.tanh(jnp.dot(adj_sc[m], sup1_sc[...],
                              preferred_element_type=jnp.float32))
        gp_bf = gp_ref[...].astype(jnp.bfloat16)
        ge = jnp.dot(gp_bf, xt.astype(jnp.bfloat16),
                     preferred_element_type=jnp.float32)
        contrib = jnp.dot(ge.astype(jnp.bfloat16), pw1_ref[...],
                          preferred_element_type=jnp.float32)
        s1 = jnp.where(m == 0, jnp.zeros_like(contrib), s1_sc[...])
        s1_sc[...] = s1 + contrib

    @pl.when(l == 0)
    def _():
        a_bf = adj_ref[...].astype(jnp.bfloat16)
        adj_sc[m] = a_bf
        xt = jnp.tanh(jnp.dot(a_bf, sup0_ref[...],
                              preferred_element_type=jnp.float32))
        xt_bf = xt.astype(jnp.bfloat16)
        x_sc[pl.ds(m * BM, BM), :] = xt_bf
        gp_bf = gp_ref[...].astype(jnp.bfloat16)
        ge = jnp.dot(gp_bf, xt_bf, preferred_element_type=jnp.float32)
        contrib = jnp.dot(ge.astype(jnp.bfloat16), pw0_ref[...],
                          preferred_element_type=jnp.float32)
        s0 = jnp.where(m == 0, jnp.zeros_like(contrib), s0_sc[...])
        s0_sc[...] = s0 + contrib

    @pl.when((l == 1) & (m == M - 1))
    def _():
        scores_ref[...] = (s0_sc[...] + s1_sc[...]
                           + jnp.broadcast_to(pb_ref[...], scores_ref.shape))


def kernel(Adj_block, X_concat, graph_pool, W0, W1, P0_w, P0_b, P1_w, P1_b):
    h0 = _hamilton(W0)
    h1 = _hamilton(W1).astype(jnp.bfloat16)
    sup0 = jnp.dot(X_concat, h0).astype(jnp.bfloat16)
    pw0 = P0_w.astype(jnp.bfloat16)
    pw1 = P1_w.astype(jnp.bfloat16)
    pb = (P0_b + P1_b).reshape(1, C)

    scores = pl.pallas_call(
        _qgnn_kernel,
        grid=(2, M),
        in_specs=[
            pl.BlockSpec((BM, N), lambda l, m: (jnp.where(l == 0, m, 0), 0)),
            pl.BlockSpec((N, H), lambda l, m: (0, 0)),
            pl.BlockSpec((G, BM), lambda l, m: (0, m)),
            pl.BlockSpec((H, H), lambda l, m: (0, 0)),
            pl.BlockSpec((H, C), lambda l, m: (0, 0)),
            pl.BlockSpec((H, C), lambda l, m: (0, 0)),
            pl.BlockSpec((1, C), lambda l, m: (0, 0)),
        ],
        out_specs=pl.BlockSpec((G, C), lambda l, m: (0, 0)),
        out_shape=jax.ShapeDtypeStruct((G, C), jnp.float32),
        scratch_shapes=[
            pltpu.VMEM((M, BM, N), jnp.bfloat16),
            pltpu.VMEM((N, H), jnp.bfloat16),
            pltpu.VMEM((N, H), jnp.bfloat16),
            pltpu.VMEM((G, C), jnp.float32),
            pltpu.VMEM((G, C), jnp.float32),
        ],
        compiler_params=pltpu.CompilerParams(
            dimension_semantics=("arbitrary", "arbitrary"),
        ),
    )(Adj_block, sup0, graph_pool, h1, pw0, pw1, pb)
    return scores
